# baseline (speedup 1.0000x reference)
import functools

import jax
import jax.numpy as jnp
from jax import lax
from jax.experimental import pallas as pl
from jax.experimental.pallas import tpu as pltpu
from jax.experimental.pallas import tpu_sc as plsc

_PART_ID = 3
_BETA = 0.25
_USAGE_REG = 0.001


def _bdot(a, b):
    return jax.lax.dot_general(
        a.astype(jnp.bfloat16), b.astype(jnp.bfloat16),
        (((1,), (0,)), ((), ())), preferred_element_type=jnp.float32)


def _enc_body(x_ref, ipW, ipb, t_ref, eW1, eb1, eW2, eb2, z_ref, *, L):
    t = t_ref[...]
    acc = None
    for l in range(L):
        h = _bdot(x_ref[l], ipW[...]) + ipb[...] + t
        h = jnp.maximum(_bdot(h, eW1[...]) + eb1[...], 0.0)
        h = jnp.maximum(_bdot(h, eW2[...]) + eb2[...], 0.0)
        acc = h if acc is None else acc + h
    z_ref[...] = acc / jnp.float32(L)


def _score_body(res_ref, q_ref, cb1_ref, cbn_ref, idx_ref, counts_ref,
                resout_ref, cacc_ref, *, NB, first):
    i = pl.program_id(0)
    if first:
        residual = res_ref[...]
    else:
        residual = res_ref[...] - q_ref[...]
    resout_ref[...] = residual

    @pl.when(i == 0)
    def _init():
        cacc_ref[...] = jnp.zeros_like(cacc_ref)

    rn = jnp.sum(residual * residual, axis=1, keepdims=True)
    scores = jax.lax.dot_general(
        residual.astype(jnp.bfloat16), cb1_ref[...],
        (((1,), (0,)), ((), ())), preferred_element_type=jnp.float32)
    d2 = rn - 2.0 * scores + cbn_ref[...]
    idx = jnp.argmin(d2, axis=1).astype(jnp.int32)
    sel = (jax.lax.broadcasted_iota(jnp.int32, d2.shape, 1) == idx[:, None])
    cacc_ref[0, :] += jnp.sum(sel.astype(jnp.float32), axis=0)
    idx_ref[0, :] = idx

    @pl.when(i == NB - 1)
    def _fin():
        counts_ref[...] = cacc_ref[...]


def _make_sc_gather(V, Dp, B):
    info = plsc.get_sparse_core_info()
    NW = info.num_cores * info.num_subcores
    b_per_w = B // NW
    mesh = plsc.VectorSubcoreMesh(core_axis_name="c", subcore_axis_name="s")

    @functools.partial(
        pl.kernel, mesh=mesh,
        out_type=jax.ShapeDtypeStruct((B, Dp), jnp.float32),
        scratch_types=[
            pltpu.VMEM((b_per_w,), jnp.int32),
            pltpu.VMEM((b_per_w, Dp), jnp.float32),
            pltpu.SemaphoreType.DMA,
        ],
    )
    def sc_gather(table_hbm, idx_hbm, out_hbm, idx_v, rows_v, sem):
        wid = lax.axis_index("s") * info.num_cores + lax.axis_index("c")
        base = wid * b_per_w
        pltpu.sync_copy(idx_hbm.at[pl.ds(base, b_per_w)], idx_v)
        pltpu.async_copy(table_hbm.at[idx_v], rows_v, sem).wait()
        pltpu.sync_copy(rows_v, out_hbm.at[pl.ds(base, b_per_w)])

    return sc_gather


def _dec_body(z_ref, q1_ref, q2_ref, q3_ref, t_ref, counts_ref,
              dW1, db1, dW2, db2, oW, ob,
              recon_ref, zq_ref, qloss_ref, usage_ref, qacc_ref,
              *, B, K, LEVELS, NB):
    i = pl.program_id(0)

    @pl.when(i == 0)
    def _init():
        qacc_ref[0, 0] = jnp.float32(0.0)

    z = z_ref[...]
    z_q = (q1_ref[...] + q2_ref[...]) + q3_ref[...]
    zst = z + (z_q - z)
    zq_ref[...] = zst
    qacc_ref[0, 0] += jnp.sum((z - z_q) ** 2)
    t = t_ref[...]
    hd = jnp.maximum(_bdot(zst + t, dW1[...]) + db1[...], 0.0)
    hd = jnp.maximum(_bdot(hd, dW2[...]) + db2[...], 0.0)
    recon_ref[...] = _bdot(hd, oW[...]) + ob[...]

    @pl.when(i == NB - 1)
    def _fin():
        probs = counts_ref[...] / jnp.float32(B)
        ent = jnp.sum(probs * jnp.log(probs + 1e-10))
        usage_ref[...] = (jnp.float32(_USAGE_REG) * (
            jnp.float32(LEVELS) * jnp.log(jnp.float32(K)) + ent)
        ).reshape(1, 1)
        qloss_ref[...] = (jnp.float32(_BETA) * qacc_ref[0, 0]
                          / jnp.float32(B * zq_ref.shape[1])).reshape(1, 1)


def kernel(x, in_proj_W, in_proj_b, type_embed, type_proj_W, type_proj_b,
           enc_W1, enc_b1, enc_W2, enc_b2, codebooks,
           dec_W1, dec_b1, dec_W2, dec_b2, out_W, out_b):
    B, L, F = x.shape
    D = in_proj_W.shape[1]
    LEVELS, K, _ = codebooks.shape
    BB = 512
    NB = B // BB
    Dp = 128

    xt = jnp.transpose(x, (1, 0, 2))
    row = lambda v: v.reshape(1, -1)
    t = (jnp.take(type_embed, _PART_ID, axis=0) @ type_proj_W
         + type_proj_b).reshape(1, D)
    cbn = jnp.sum(codebooks * codebooks, axis=2)
    cb1 = jnp.transpose(codebooks, (0, 2, 1)).astype(jnp.bfloat16)
    cbpad = jnp.pad(codebooks, ((0, 0), (0, 0), (0, Dp - D)))

    const = lambda shape: pl.BlockSpec(shape, lambda i: tuple(0 for _ in shape))

    z = pl.pallas_call(
        functools.partial(_enc_body, L=L),
        grid=(NB,),
        in_specs=[
            pl.BlockSpec((L, BB, F), lambda i: (0, i, 0)),
            const((F, D)), const((1, D)), const((1, D)),
            const((D, D)), const((1, D)), const((D, D)), const((1, D)),
        ],
        out_specs=pl.BlockSpec((BB, D), lambda i: (i, 0)),
        out_shape=jax.ShapeDtypeStruct((B, D), jnp.float32),
    )(xt, in_proj_W, row(in_proj_b), t,
      enc_W1, row(enc_b1), enc_W2, row(enc_b2))

    sc_gather = _make_sc_gather(K, Dp, B)
    score_call = lambda first: pl.pallas_call(
        functools.partial(_score_body, NB=NB, first=first),
        grid=(NB,),
        in_specs=[
            pl.BlockSpec((BB, D), lambda i: (i, 0)),
            pl.BlockSpec((BB, D), lambda i: (i, 0)),
            const((D, K)), const((1, K)),
        ],
        out_specs=[
            pl.BlockSpec((1, BB), lambda i: (0, i)),
            pl.BlockSpec((1, K), lambda i: (0, 0)),
            pl.BlockSpec((BB, D), lambda i: (i, 0)),
        ],
        out_shape=[
            jax.ShapeDtypeStruct((1, B), jnp.int32),
            jax.ShapeDtypeStruct((1, K), jnp.float32),
            jax.ShapeDtypeStruct((B, D), jnp.float32),
        ],
        scratch_shapes=[pltpu.VMEM((1, K), jnp.float32)],
    )

    prev = z
    q_prev = z
    idx_rows, counts_rows, qs = [], [], []
    for lvl in range(LEVELS):
        idx_row, counts_l, residual = score_call(lvl == 0)(
            prev, q_prev, cb1[lvl], cbn[lvl].reshape(1, K))
        q128 = sc_gather(cbpad[lvl], idx_row.reshape(B))
        q = q128[:, :D]
        idx_rows.append(idx_row)
        counts_rows.append(counts_l)
        qs.append(q)
        prev, q_prev = residual, q

    counts = jnp.concatenate(counts_rows, axis=0)
    codesT = jnp.concatenate(idx_rows, axis=0)

    recon, zqst, qloss, usage = pl.pallas_call(
        functools.partial(_dec_body, B=B, K=K, LEVELS=LEVELS, NB=NB),
        grid=(NB,),
        in_specs=[
            pl.BlockSpec((BB, D), lambda i: (i, 0)),
            pl.BlockSpec((BB, D), lambda i: (i, 0)),
            pl.BlockSpec((BB, D), lambda i: (i, 0)),
            pl.BlockSpec((BB, D), lambda i: (i, 0)),
            const((1, D)), const((LEVELS, K)),
            const((D, D)), const((1, D)), const((D, D)), const((1, D)),
            const((D, L * F)), const((1, L * F)),
        ],
        out_specs=[
            pl.BlockSpec((BB, L * F), lambda i: (i, 0)),
            pl.BlockSpec((BB, D), lambda i: (i, 0)),
            pl.BlockSpec((1, 1), lambda i: (0, 0)),
            pl.BlockSpec((1, 1), lambda i: (0, 0)),
        ],
        out_shape=[
            jax.ShapeDtypeStruct((B, L * F), jnp.float32),
            jax.ShapeDtypeStruct((B, D), jnp.float32),
            jax.ShapeDtypeStruct((1, 1), jnp.float32),
            jax.ShapeDtypeStruct((1, 1), jnp.float32),
        ],
        scratch_shapes=[pltpu.SMEM((1, 1), jnp.float32)],
    )(z, qs[0], qs[1], qs[2], t, counts,
      dec_W1, row(dec_b1), dec_W2, row(dec_b2), out_W, row(out_b))

    return (recon.reshape(B, L, F), codesT.T, qloss[0, 0], usage[0, 0],
            zqst)


# speedup vs baseline: 1.0006x; 1.0006x over previous
import functools

import jax
import jax.numpy as jnp
from jax import lax
from jax.experimental import pallas as pl
from jax.experimental.pallas import tpu as pltpu
from jax.experimental.pallas import tpu_sc as plsc

_PART_ID = 3
_BETA = 0.25
_USAGE_REG = 0.001


def _bdot(a, b):
    return jax.lax.dot_general(
        a.astype(jnp.bfloat16), b.astype(jnp.bfloat16),
        (((1,), (0,)), ((), ())), preferred_element_type=jnp.float32)


def _enc_body(x_ref, ipW, ipb, t_ref, eW1, eb1, eW2, eb2, z_ref, *, L):
    t = t_ref[...]
    acc = None
    for l in range(L):
        h = _bdot(x_ref[l], ipW[...]) + ipb[...] + t
        h = jnp.maximum(_bdot(h, eW1[...]) + eb1[...], 0.0)
        h = jnp.maximum(_bdot(h, eW2[...]) + eb2[...], 0.0)
        acc = h if acc is None else acc + h
    z_ref[...] = acc / jnp.float32(L)


def _score_body(res_ref, q_ref, cb1_ref, cbn_ref, idx_ref, counts_ref,
                resout_ref, cacc_ref, *, NB, first):
    i = pl.program_id(0)
    if first:
        residual = res_ref[...]
    else:
        residual = res_ref[...] - q_ref[:, :res_ref.shape[1]]
    resout_ref[...] = residual

    @pl.when(i == 0)
    def _init():
        cacc_ref[...] = jnp.zeros_like(cacc_ref)

    rn = jnp.sum(residual * residual, axis=1, keepdims=True)
    scores = jax.lax.dot_general(
        residual.astype(jnp.bfloat16), cb1_ref[...],
        (((1,), (0,)), ((), ())), preferred_element_type=jnp.float32)
    d2 = rn - 2.0 * scores + cbn_ref[...]
    idx = jnp.argmin(d2, axis=1).astype(jnp.int32)
    sel = (jax.lax.broadcasted_iota(jnp.int32, d2.shape, 1) == idx[:, None])
    cacc_ref[0, :] += jnp.sum(sel.astype(jnp.float32), axis=0)
    idx_ref[0, :] = idx

    @pl.when(i == NB - 1)
    def _fin():
        counts_ref[...] = cacc_ref[...]


def _make_sc_gather(V, Dp, B):
    info = plsc.get_sparse_core_info()
    NW = info.num_cores * info.num_subcores
    b_per_w = B // NW
    mesh = plsc.VectorSubcoreMesh(core_axis_name="c", subcore_axis_name="s")

    @functools.partial(
        pl.kernel, mesh=mesh,
        out_type=jax.ShapeDtypeStruct((B, Dp), jnp.float32),
        scratch_types=[
            pltpu.VMEM((b_per_w,), jnp.int32),
            pltpu.VMEM((b_per_w, Dp), jnp.float32),
            pltpu.SemaphoreType.DMA,
        ],
    )
    def sc_gather(table_hbm, idx_hbm, out_hbm, idx_v, rows_v, sem):
        wid = lax.axis_index("s") * info.num_cores + lax.axis_index("c")
        base = wid * b_per_w
        pltpu.sync_copy(idx_hbm.at[pl.ds(base, b_per_w)], idx_v)
        pltpu.async_copy(table_hbm.at[idx_v], rows_v, sem).wait()
        pltpu.sync_copy(rows_v, out_hbm.at[pl.ds(base, b_per_w)])

    return sc_gather


def _dec_body(z_ref, q1_ref, q2_ref, q3_ref, t_ref, counts_ref,
              dW1, db1, dW2, db2, oW, ob,
              recon_ref, zq_ref, qloss_ref, usage_ref, qacc_ref,
              *, B, K, LEVELS, NB):
    i = pl.program_id(0)

    @pl.when(i == 0)
    def _init():
        qacc_ref[0, 0] = jnp.float32(0.0)

    z = z_ref[...]
    D = z.shape[1]
    z_q = (q1_ref[:, :D] + q2_ref[:, :D]) + q3_ref[:, :D]
    zst = z + (z_q - z)
    zq_ref[...] = zst
    qacc_ref[0, 0] += jnp.sum((z - z_q) ** 2)
    t = t_ref[...]
    hd = jnp.maximum(_bdot(zst + t, dW1[...]) + db1[...], 0.0)
    hd = jnp.maximum(_bdot(hd, dW2[...]) + db2[...], 0.0)
    recon_ref[...] = _bdot(hd, oW[...]) + ob[...]

    @pl.when(i == NB - 1)
    def _fin():
        probs = counts_ref[...] / jnp.float32(B)
        ent = jnp.sum(probs * jnp.log(probs + 1e-10))
        usage_ref[...] = (jnp.float32(_USAGE_REG) * (
            jnp.float32(LEVELS) * jnp.log(jnp.float32(K)) + ent)
        ).reshape(1, 1)
        qloss_ref[...] = (jnp.float32(_BETA) * qacc_ref[0, 0]
                          / jnp.float32(B * zq_ref.shape[1])).reshape(1, 1)


def kernel(x, in_proj_W, in_proj_b, type_embed, type_proj_W, type_proj_b,
           enc_W1, enc_b1, enc_W2, enc_b2, codebooks,
           dec_W1, dec_b1, dec_W2, dec_b2, out_W, out_b):
    B, L, F = x.shape
    D = in_proj_W.shape[1]
    LEVELS, K, _ = codebooks.shape
    BB = 512
    NB = B // BB
    Dp = 128

    xt = jnp.transpose(x, (1, 0, 2))
    row = lambda v: v.reshape(1, -1)
    t = (jnp.take(type_embed, _PART_ID, axis=0) @ type_proj_W
         + type_proj_b).reshape(1, D)
    cbn = jnp.sum(codebooks * codebooks, axis=2)
    cb1 = jnp.transpose(codebooks, (0, 2, 1)).astype(jnp.bfloat16)
    cbpad = jnp.pad(codebooks, ((0, 0), (0, 0), (0, Dp - D)))

    const = lambda shape: pl.BlockSpec(shape, lambda i: tuple(0 for _ in shape))

    z = pl.pallas_call(
        functools.partial(_enc_body, L=L),
        grid=(NB,),
        in_specs=[
            pl.BlockSpec((L, BB, F), lambda i: (0, i, 0)),
            const((F, D)), const((1, D)), const((1, D)),
            const((D, D)), const((1, D)), const((D, D)), const((1, D)),
        ],
        out_specs=pl.BlockSpec((BB, D), lambda i: (i, 0)),
        out_shape=jax.ShapeDtypeStruct((B, D), jnp.float32),
    )(xt, in_proj_W, row(in_proj_b), t,
      enc_W1, row(enc_b1), enc_W2, row(enc_b2))

    sc_gather = _make_sc_gather(K, Dp, B)
    score_call = lambda first: pl.pallas_call(
        functools.partial(_score_body, NB=NB, first=first),
        grid=(NB,),
        in_specs=[
            pl.BlockSpec((BB, D), lambda i: (i, 0)),
            pl.BlockSpec((BB, D if first else Dp), lambda i: (i, 0)),
            const((D, K)), const((1, K)),
        ],
        out_specs=[
            pl.BlockSpec((1, BB), lambda i: (0, i)),
            pl.BlockSpec((1, K), lambda i: (0, 0)),
            pl.BlockSpec((BB, D), lambda i: (i, 0)),
        ],
        out_shape=[
            jax.ShapeDtypeStruct((1, B), jnp.int32),
            jax.ShapeDtypeStruct((1, K), jnp.float32),
            jax.ShapeDtypeStruct((B, D), jnp.float32),
        ],
        scratch_shapes=[pltpu.VMEM((1, K), jnp.float32)],
    )

    prev = z
    q_prev = z
    idx_rows, counts_rows, qs = [], [], []
    for lvl in range(LEVELS):
        idx_row, counts_l, residual = score_call(lvl == 0)(
            prev, q_prev, cb1[lvl], cbn[lvl].reshape(1, K))
        q = sc_gather(cbpad[lvl], idx_row.reshape(B))
        idx_rows.append(idx_row)
        counts_rows.append(counts_l)
        qs.append(q)
        prev, q_prev = residual, q

    counts = jnp.concatenate(counts_rows, axis=0)
    codesT = jnp.concatenate(idx_rows, axis=0)

    recon, zqst, qloss, usage = pl.pallas_call(
        functools.partial(_dec_body, B=B, K=K, LEVELS=LEVELS, NB=NB),
        grid=(NB,),
        in_specs=[
            pl.BlockSpec((BB, D), lambda i: (i, 0)),
            pl.BlockSpec((BB, Dp), lambda i: (i, 0)),
            pl.BlockSpec((BB, Dp), lambda i: (i, 0)),
            pl.BlockSpec((BB, Dp), lambda i: (i, 0)),
            const((1, D)), const((LEVELS, K)),
            const((D, D)), const((1, D)), const((D, D)), const((1, D)),
            const((D, L * F)), const((1, L * F)),
        ],
        out_specs=[
            pl.BlockSpec((BB, L * F), lambda i: (i, 0)),
            pl.BlockSpec((BB, D), lambda i: (i, 0)),
            pl.BlockSpec((1, 1), lambda i: (0, 0)),
            pl.BlockSpec((1, 1), lambda i: (0, 0)),
        ],
        out_shape=[
            jax.ShapeDtypeStruct((B, L * F), jnp.float32),
            jax.ShapeDtypeStruct((B, D), jnp.float32),
            jax.ShapeDtypeStruct((1, 1), jnp.float32),
            jax.ShapeDtypeStruct((1, 1), jnp.float32),
        ],
        scratch_shapes=[pltpu.SMEM((1, 1), jnp.float32)],
    )(z, qs[0], qs[1], qs[2], t, counts,
      dec_W1, row(dec_b1), dec_W2, row(dec_b2), out_W, row(out_b))

    return (recon.reshape(B, L, F), codesT.T, qloss[0, 0], usage[0, 0],
            zqst)


# speedup vs baseline: 2.0098x; 2.0086x over previous
import functools

import jax
import jax.numpy as jnp
from jax.experimental import pallas as pl
from jax.experimental.pallas import tpu as pltpu

_PART_ID = 3
_BETA = 0.25
_USAGE_REG = 0.001
_HI = jax.lax.Precision.HIGHEST


def _bdot(a, b, dims):
    return jax.lax.dot_general(
        a.astype(jnp.bfloat16), b.astype(jnp.bfloat16), (dims, ((), ())),
        preferred_element_type=jnp.float32)


def _body(x_ref, ipW, ipb, t_ref, eW1, eb1, eW2, eb2,
          cbc_ref, cbn_ref,
          dW1, db1, dW2, db2, oW, ob,
          recon_ref, codesT_ref, qloss_ref, usage_ref, zq_ref,
          counts_ref, qacc_ref, *, L, K, LEVELS, B, NB):
    i = pl.program_id(0)

    @pl.when(i == 0)
    def _init():
        counts_ref[...] = jnp.zeros_like(counts_ref)
        qacc_ref[0, 0] = jnp.float32(0.0)

    t = t_ref[...]

    acc = None
    for l in range(L):
        xl = x_ref[l]
        h = _bdot(xl, ipW[...], ((1,), (0,))) + ipb[...] + t
        h = jnp.maximum(_bdot(h, eW1[...], ((1,), (0,))) + eb1[...], 0.0)
        h = jnp.maximum(_bdot(h, eW2[...], ((1,), (0,))) + eb2[...], 0.0)
        acc = h if acc is None else acc + h
    z = acc / jnp.float32(L)

    residual = z
    z_q = jnp.zeros_like(z)
    D = zq_ref.shape[1]
    for lvl in range(LEVELS):
        cbc = cbc_ref[lvl]
        cbn = cbn_ref[lvl]
        rn = jnp.sum(residual * residual, axis=1, keepdims=True)
        scores = jax.lax.dot_general(
            residual.astype(jnp.bfloat16), cbc[0:D, :],
            ((((1,), (0,))), ((), ())),
            preferred_element_type=jnp.float32)
        d2 = rn - 2.0 * scores + cbn[None, :]
        idx = jnp.argmin(d2, axis=1).astype(jnp.int32)
        sel = (jax.lax.broadcasted_iota(jnp.int32, d2.shape, 1)
               == idx[:, None])
        onehot = sel.astype(jnp.bfloat16)
        qcat = jax.lax.dot_general(
            onehot, cbc, (((1,), (1,)), ((), ())),
            preferred_element_type=jnp.float32)
        q = (qcat[:, 0:D] + qcat[:, D:2 * D]) + qcat[:, 2 * D:3 * D]
        z_q = z_q + q
        residual = residual - q
        counts_ref[lvl, :] += jnp.sum(sel.astype(jnp.float32), axis=0)
        codesT_ref[lvl, :] = idx

    zst = z + (z_q - z)
    zq_ref[...] = zst
    qacc_ref[0, 0] += jnp.sum((z - z_q) ** 2)

    hd = jnp.maximum(_bdot(zst + t, dW1[...], ((1,), (0,))) + db1[...], 0.0)
    hd = jnp.maximum(_bdot(hd, dW2[...], ((1,), (0,))) + db2[...], 0.0)
    recon_ref[...] = _bdot(hd, oW[...], ((1,), (0,))) + ob[...]

    @pl.when(i == NB - 1)
    def _finish():
        probs = counts_ref[...] / jnp.float32(B)
        ent = jnp.sum(probs * jnp.log(probs + 1e-10))
        usage_ref[...] = (jnp.float32(_USAGE_REG) * (
            jnp.float32(LEVELS) * jnp.log(jnp.float32(K)) + ent)
        ).reshape(1, 1)
        qloss_ref[...] = (jnp.float32(_BETA) * qacc_ref[0, 0]
                          / jnp.float32(B * zq_ref.shape[1])).reshape(1, 1)


def kernel(x, in_proj_W, in_proj_b, type_embed, type_proj_W, type_proj_b,
           enc_W1, enc_b1, enc_W2, enc_b2, codebooks,
           dec_W1, dec_b1, dec_W2, dec_b2, out_W, out_b):
    B, L, F = x.shape
    D = in_proj_W.shape[1]
    LEVELS, K, _ = codebooks.shape
    BB = 512
    NB = B // BB

    xt = jnp.transpose(x, (1, 0, 2))
    row = lambda v: v.reshape(1, -1)
    t = (jnp.take(type_embed, _PART_ID, axis=0) @ type_proj_W
         + type_proj_b).reshape(1, D)
    cbn = jnp.sum(codebooks * codebooks, axis=2)
    cbT = jnp.transpose(codebooks, (0, 2, 1))
    cb1 = cbT.astype(jnp.bfloat16)
    r1 = cbT - jax.lax.optimization_barrier(cb1).astype(jnp.float32)
    cb2 = r1.astype(jnp.bfloat16)
    cb3 = (r1 - jax.lax.optimization_barrier(cb2).astype(jnp.float32)
           ).astype(jnp.bfloat16)
    cbc = jnp.concatenate([cb1, cb2, cb3], axis=1)

    const = lambda shape: pl.BlockSpec(shape, lambda i: tuple(0 for _ in shape))

    grid_spec = pltpu.PrefetchScalarGridSpec(
        num_scalar_prefetch=0,
        grid=(NB,),
        in_specs=[
            pl.BlockSpec((L, BB, F), lambda i: (0, i, 0)),
            const((F, D)), const((1, D)), const((1, D)),
            const((D, D)), const((1, D)), const((D, D)), const((1, D)),
            const((LEVELS, 3 * D, K)), const((LEVELS, K)),
            const((D, D)), const((1, D)), const((D, D)), const((1, D)),
            const((D, L * F)), const((1, L * F)),
        ],
        out_specs=[
            pl.BlockSpec((BB, L * F), lambda i: (i, 0)),
            pl.BlockSpec((LEVELS, BB), lambda i: (0, i)),
            pl.BlockSpec((1, 1), lambda i: (0, 0)),
            pl.BlockSpec((1, 1), lambda i: (0, 0)),
            pl.BlockSpec((BB, D), lambda i: (i, 0)),
        ],
        scratch_shapes=[
            pltpu.VMEM((LEVELS, K), jnp.float32),
            pltpu.SMEM((1, 1), jnp.float32),
        ],
    )

    recon, codesT, qloss, usage, zqst = pl.pallas_call(
        functools.partial(_body, L=L, K=K, LEVELS=LEVELS, B=B, NB=NB),
        grid_spec=grid_spec,
        out_shape=[
            jax.ShapeDtypeStruct((B, L * F), jnp.float32),
            jax.ShapeDtypeStruct((LEVELS, B), jnp.int32),
            jax.ShapeDtypeStruct((1, 1), jnp.float32),
            jax.ShapeDtypeStruct((1, 1), jnp.float32),
            jax.ShapeDtypeStruct((B, D), jnp.float32),
        ],
    )(xt, in_proj_W, row(in_proj_b), t,
      enc_W1, row(enc_b1), enc_W2, row(enc_b2),
      cbc, cbn,
      dec_W1, row(dec_b1), dec_W2, row(dec_b2),
      out_W, row(out_b))

    return (recon.reshape(B, L, F), codesT.T, qloss[0, 0], usage[0, 0],
            zqst)
